# baseline (speedup 1.0000x reference)
import functools

import jax
import jax.numpy as jnp
from jax.experimental import pallas as pl
from jax.experimental.pallas import tpu as pltpu
from jax.experimental.pallas import tpu_sc as plsc

_N = 20000
_TOP_K = 50
_IOU_THR = 0.3
_V0 = 0.1
_V1 = 0.2
_NS = 16
_W = 1280
_PAD = _NS * _W
_NVREG = _W // 16
_NEG_INF = float('-inf')


def _decode_kernel(loc_ref, conf_ref, iou_ref, pri_ref, out_ref):
    pcx = pri_ref[0]
    pcy = pri_ref[1]
    pw = pri_ref[2]
    ph = pri_ref[3]

    rowi = jax.lax.broadcasted_iota(jnp.int32, (_NS, _W), 0)
    coli = jax.lax.broadcasted_iota(jnp.int32, (_NS, _W), 1)
    lin = rowi * _W + coli

    cls = conf_ref[...]
    iouc = jnp.clip(iou_ref[...], 0.0, 1.0)
    scores = jnp.sqrt(cls * iouc)
    out_ref[0] = jnp.where(lin < _N, scores, _NEG_INF)

    cx = pcx + loc_ref[0] * _V0 * pw
    cy = pcy + loc_ref[1] * _V0 * ph
    wx = pw * jnp.exp(loc_ref[2] * _V0) * 0.5
    hy = ph * jnp.exp(loc_ref[3] * _V1) * 0.5
    x1 = cx - wx
    y1 = cy - hy
    x2 = cx + wx
    y2 = cy + hy
    out_ref[1] = x1
    out_ref[2] = y1
    out_ref[3] = x2
    out_ref[4] = y2
    out_ref[5] = (x2 - x1) * (y2 - y1)
    for k in range(5):
        out_ref[6 + 2 * k] = pcx + loc_ref[4 + 2 * k] * _V0 * pw
        out_ref[7 + 2 * k] = pcy + loc_ref[5 + 2 * k] * _V0 * ph


def _nms_sc_kernel(planes_hbm, out_hbm,
                   s_v, s0_v, x1_v, y1_v, x2_v, y2_v, ar_v, lm_v,
                   pub_v, shloc_v, row_v, sh, rows_sh):
    c_id = jax.lax.axis_index("c")
    s_id = jax.lax.axis_index("s")

    pltpu.sync_copy(planes_hbm.at[0, s_id], s_v)
    pltpu.sync_copy(planes_hbm.at[0, s_id], s0_v)
    pltpu.sync_copy(planes_hbm.at[1, s_id], x1_v)
    pltpu.sync_copy(planes_hbm.at[2, s_id], y1_v)
    pltpu.sync_copy(planes_hbm.at[3, s_id], x2_v)
    pltpu.sync_copy(planes_hbm.at[4, s_id], y2_v)
    pltpu.sync_copy(planes_hbm.at[5, s_id], ar_v)
    for k in range(10):
        pltpu.sync_copy(planes_hbm.at[6 + k, s_id],
                        lm_v.at[pl.ds(k * _W, _W)])

    iota16 = jax.lax.iota(jnp.int32, 16)
    zero16 = jnp.zeros((16,), jnp.int32)

    def splat_at(ref, base, lane):
        chunk = ref[pl.ds(base, 16)]
        return chunk[lane + zero16]

    bestv0 = s_v[pl.ds(0, 16)]
    bestj0 = zero16
    for j in range(1, _NVREG):
        v = s_v[pl.ds(16 * j, 16)]
        take = v > bestv0
        bestv0 = jnp.where(take, v, bestv0)
        bestj0 = jnp.where(take, j, bestj0)

    def round_body(i, carry):
        bestv, bestj = carry
        mv = bestv
        for k in (8, 4, 2, 1):
            mv = jnp.maximum(mv, mv[jnp.bitwise_xor(iota16, k)])
        cand = jnp.where(bestv == mv, bestj * 16 + iota16, jnp.int32(1 << 30))
        for k in (8, 4, 2, 1):
            cand = jnp.minimum(cand, cand[jnp.bitwise_xor(iota16, k)])
        lj = cand[0]
        base = jnp.bitwise_and(lj, -16)
        lane = jnp.bitwise_and(lj, 15)
        gidxf = (s_id * _W + lj).astype(jnp.float32)

        pub = jnp.where(iota16 == 0, mv, 0.0)
        pub = jnp.where(iota16 == 1, gidxf, pub)
        pub = jnp.where(iota16 == 2, splat_at(x1_v, base, lane), pub)
        pub = jnp.where(iota16 == 3, splat_at(y1_v, base, lane), pub)
        pub = jnp.where(iota16 == 4, splat_at(x2_v, base, lane), pub)
        pub = jnp.where(iota16 == 5, splat_at(y2_v, base, lane), pub)
        pub = jnp.where(iota16 == 6, splat_at(ar_v, base, lane), pub)
        pub = jnp.where(iota16 == 7, s_id.astype(jnp.float32), pub)
        pub_v[...] = pub
        par = jnp.bitwise_and(i, 1) * 256
        pltpu.sync_copy(pub_v, sh.at[pl.ds(par + 16 * s_id, 16)])
        plsc.subcore_barrier()

        pltpu.sync_copy(sh.at[pl.ds(par, 256)], shloc_v)

        def merge(a, b):
            va = a[0]
            vb = b[0]
            cond = jnp.logical_or(
                vb > va, jnp.logical_and(vb == va, b[1] < a[1]))
            return jnp.where(cond, b, a)

        rows = [shloc_v[pl.ds(16 * w, 16)] for w in range(_NS)]
        while len(rows) > 1:
            rows = [merge(rows[2 * k], rows[2 * k + 1])
                    for k in range(len(rows) // 2)]
        best = rows[0]
        bx1 = best[2]
        by1 = best[3]
        bx2 = best[4]
        by2 = best[5]
        barea = best[6]
        w_id = best[7].astype(jnp.int32)

        @pl.when(jnp.logical_and(w_id == s_id, c_id == 0))
        def _():
            ljw = best[1].astype(jnp.int32) - s_id * _W
            wbase = jnp.bitwise_and(ljw, -16)
            wlane = jnp.bitwise_and(ljw, 15)
            row = jnp.where(iota16 == 0, bx1, 0.0)
            row = jnp.where(iota16 == 1, by1, row)
            row = jnp.where(iota16 == 2, bx2, row)
            row = jnp.where(iota16 == 3, by2, row)
            for k in range(10):
                lmv = splat_at(lm_v, k * _W + wbase, wlane)
                row = jnp.where(iota16 == 4 + k, lmv, row)
            row = jnp.where(iota16 == 14, splat_at(s0_v, wbase, wlane), row)
            row_v[...] = row
            pltpu.sync_copy(row_v, rows_sh.at[i])

        nbv = jnp.full((16,), _NEG_INF, jnp.float32)
        nbj = zero16
        for j in range(_NVREG):
            ds = pl.ds(16 * j, 16)
            ix1 = jnp.maximum(bx1, x1_v[ds])
            iy1 = jnp.maximum(by1, y1_v[ds])
            ix2 = jnp.minimum(bx2, x2_v[ds])
            iy2 = jnp.minimum(by2, y2_v[ds])
            inter = (jnp.maximum(ix2 - ix1, 0.0)
                     * jnp.maximum(iy2 - iy1, 0.0))
            union = barea + ar_v[ds] - inter
            iouv = inter / jnp.maximum(union, 1e-12)
            ns = jnp.where(iouv <= _IOU_THR, s_v[ds], _NEG_INF)
            s_v[ds] = ns
            take = ns > nbv
            nbv = jnp.where(take, ns, nbv)
            nbj = jnp.where(take, j, nbj)

        return nbv, nbj

    jax.lax.fori_loop(0, _TOP_K, round_body, (bestv0, bestj0))

    plsc.subcore_barrier()
    @pl.when(jnp.logical_and(s_id == 0, c_id == 0))
    def _():
        pltpu.sync_copy(rows_sh, out_hbm)


def _plane(x):
    return jnp.pad(x, (0, _PAD - _N)).reshape(_NS, _W)


@jax.jit
def kernel(loc, conf, iou, priors):
    loc_p = jnp.stack([_plane(loc[:, k]) for k in range(14)])
    conf_p = _plane(conf[:, 1])
    iou_p = _plane(iou[:, 0])
    pri_p = jnp.stack([_plane(priors[:, k]) for k in range(4)])

    planes = pl.pallas_call(
        _decode_kernel,
        out_shape=jax.ShapeDtypeStruct((16, _NS, _W), jnp.float32),
    )(loc_p, conf_p, iou_p, pri_p)

    mesh = plsc.VectorSubcoreMesh(core_axis_name="c", subcore_axis_name="s",
                                  num_cores=2, num_subcores=_NS)
    nms = functools.partial(
        pl.kernel,
        out_type=jax.ShapeDtypeStruct((_TOP_K, 16), jnp.float32),
        mesh=mesh,
        scratch_types=[
            pltpu.VMEM((_W,), jnp.float32),
            pltpu.VMEM((_W,), jnp.float32),
            pltpu.VMEM((_W,), jnp.float32),
            pltpu.VMEM((_W,), jnp.float32),
            pltpu.VMEM((_W,), jnp.float32),
            pltpu.VMEM((_W,), jnp.float32),
            pltpu.VMEM((_W,), jnp.float32),
            pltpu.VMEM((10 * _W,), jnp.float32),
            pltpu.VMEM((16,), jnp.float32),
            pltpu.VMEM((16 * 16,), jnp.float32),
            pltpu.VMEM((16,), jnp.float32),
            pltpu.VMEM_SHARED((2 * 16 * 16,), jnp.float32),
            pltpu.VMEM_SHARED((_TOP_K, 16), jnp.float32),
        ],
    )(_nms_sc_kernel)

    out = nms(planes)
    return out[:, :15]


# speedup vs baseline: 1.0035x; 1.0035x over previous
import functools

import jax
import jax.numpy as jnp
from jax.experimental import pallas as pl
from jax.experimental.pallas import tpu as pltpu
from jax.experimental.pallas import tpu_sc as plsc

_N = 20000
_TOP_K = 50
_IOU_THR = 0.3
_V0 = 0.1
_V1 = 0.2
_NS = 16
_W = 1280
_PAD = _NS * _W
_NVREG = _W // 16
_NEG_INF = float('-inf')


def _decode_kernel(loc_ref, conf_ref, iou_ref, pri_ref, out_ref):
    pcx = pri_ref[0]
    pcy = pri_ref[1]
    pw = pri_ref[2]
    ph = pri_ref[3]

    rowi = jax.lax.broadcasted_iota(jnp.int32, (_NS, _W), 0)
    coli = jax.lax.broadcasted_iota(jnp.int32, (_NS, _W), 1)
    lin = rowi * _W + coli

    cls = conf_ref[...]
    iouc = jnp.clip(iou_ref[...], 0.0, 1.0)
    scores = jnp.sqrt(cls * iouc)
    out_ref[0] = jnp.where(lin < _N, scores, _NEG_INF)

    cx = pcx + loc_ref[0] * _V0 * pw
    cy = pcy + loc_ref[1] * _V0 * ph
    wx = pw * jnp.exp(loc_ref[2] * _V0) * 0.5
    hy = ph * jnp.exp(loc_ref[3] * _V1) * 0.5
    x1 = cx - wx
    y1 = cy - hy
    x2 = cx + wx
    y2 = cy + hy
    out_ref[1] = x1
    out_ref[2] = y1
    out_ref[3] = x2
    out_ref[4] = y2
    out_ref[5] = (x2 - x1) * (y2 - y1)
    for k in range(5):
        out_ref[6 + 2 * k] = pcx + loc_ref[4 + 2 * k] * _V0 * pw
        out_ref[7 + 2 * k] = pcy + loc_ref[5 + 2 * k] * _V0 * ph


def _nms_sc_kernel(planes_hbm, out_hbm,
                   s_v, s0_v, x1_v, y1_v, x2_v, y2_v, ar_v, lm_v,
                   pub_v, shloc_v, row_v, sh, rows_sh):
    c_id = jax.lax.axis_index("c")
    s_id = jax.lax.axis_index("s")

    pltpu.sync_copy(planes_hbm.at[0, s_id], s_v)
    pltpu.sync_copy(planes_hbm.at[0, s_id], s0_v)
    pltpu.sync_copy(planes_hbm.at[1, s_id], x1_v)
    pltpu.sync_copy(planes_hbm.at[2, s_id], y1_v)
    pltpu.sync_copy(planes_hbm.at[3, s_id], x2_v)
    pltpu.sync_copy(planes_hbm.at[4, s_id], y2_v)
    pltpu.sync_copy(planes_hbm.at[5, s_id], ar_v)
    for k in range(10):
        pltpu.sync_copy(planes_hbm.at[6 + k, s_id],
                        lm_v.at[pl.ds(k * _W, _W)])

    iota16 = jax.lax.iota(jnp.int32, 16)
    zero16 = jnp.zeros((16,), jnp.int32)

    def splat_at(ref, base, lane):
        chunk = ref[pl.ds(base, 16)]
        return chunk[lane + zero16]

    bestv0 = s_v[pl.ds(0, 16)]
    bestj0 = zero16
    for j in range(1, _NVREG):
        v = s_v[pl.ds(16 * j, 16)]
        take = v > bestv0
        bestv0 = jnp.where(take, v, bestv0)
        bestj0 = jnp.where(take, j, bestj0)

    def round_body(i, carry):
        bestv, bestj = carry
        mv = bestv
        for k in (8, 4, 2, 1):
            mv = jnp.maximum(mv, mv[jnp.bitwise_xor(iota16, k)])
        cand = jnp.where(bestv == mv, bestj * 16 + iota16, jnp.int32(1 << 30))
        for k in (8, 4, 2, 1):
            cand = jnp.minimum(cand, cand[jnp.bitwise_xor(iota16, k)])
        lj = cand[0]
        base = jnp.bitwise_and(lj, -16)
        lane = jnp.bitwise_and(lj, 15)
        gidxf = (s_id * _W + lj).astype(jnp.float32)

        pub = jnp.where(iota16 == 0, mv, 0.0)
        pub = jnp.where(iota16 == 1, gidxf, pub)
        pub = jnp.where(iota16 == 2, splat_at(x1_v, base, lane), pub)
        pub = jnp.where(iota16 == 3, splat_at(y1_v, base, lane), pub)
        pub = jnp.where(iota16 == 4, splat_at(x2_v, base, lane), pub)
        pub = jnp.where(iota16 == 5, splat_at(y2_v, base, lane), pub)
        pub = jnp.where(iota16 == 6, splat_at(ar_v, base, lane), pub)
        pub = jnp.where(iota16 == 7, s_id.astype(jnp.float32), pub)
        pub_v[...] = pub
        par = jnp.bitwise_and(i, 1) * 256
        pltpu.sync_copy(pub_v, sh.at[pl.ds(par + 16 * s_id, 16)])
        plsc.subcore_barrier()

        pltpu.sync_copy(sh.at[pl.ds(par, 256)], shloc_v)

        def merge(a, b):
            va = a[0]
            vb = b[0]
            cond = jnp.logical_or(
                vb > va, jnp.logical_and(vb == va, b[1] < a[1]))
            return jnp.where(cond, b, a)

        rows = [shloc_v[pl.ds(16 * w, 16)] for w in range(_NS)]
        while len(rows) > 1:
            rows = [merge(rows[2 * k], rows[2 * k + 1])
                    for k in range(len(rows) // 2)]
        best = rows[0]
        bx1 = best[2]
        by1 = best[3]
        bx2 = best[4]
        by2 = best[5]
        barea = best[6]
        w_id = best[7].astype(jnp.int32)

        @pl.when(jnp.logical_and(w_id == s_id, c_id == 0))
        def _():
            ljw = best[1].astype(jnp.int32) - s_id * _W
            wbase = jnp.bitwise_and(ljw, -16)
            wlane = jnp.bitwise_and(ljw, 15)
            row = jnp.where(iota16 == 0, bx1, 0.0)
            row = jnp.where(iota16 == 1, by1, row)
            row = jnp.where(iota16 == 2, bx2, row)
            row = jnp.where(iota16 == 3, by2, row)
            for k in range(10):
                lmv = splat_at(lm_v, k * _W + wbase, wlane)
                row = jnp.where(iota16 == 4 + k, lmv, row)
            row = jnp.where(iota16 == 14, splat_at(s0_v, wbase, wlane), row)
            row_v[...] = row
            pltpu.sync_copy(row_v, rows_sh.at[pl.ds(16 * i, 16)])

        nbv = jnp.full((16,), _NEG_INF, jnp.float32)
        nbj = zero16
        for j in range(_NVREG):
            ds = pl.ds(16 * j, 16)
            ix1 = jnp.maximum(bx1, x1_v[ds])
            iy1 = jnp.maximum(by1, y1_v[ds])
            ix2 = jnp.minimum(bx2, x2_v[ds])
            iy2 = jnp.minimum(by2, y2_v[ds])
            inter = (jnp.maximum(ix2 - ix1, 0.0)
                     * jnp.maximum(iy2 - iy1, 0.0))
            union = barea + ar_v[ds] - inter
            iouv = inter / jnp.maximum(union, 1e-12)
            ns = jnp.where(iouv <= _IOU_THR, s_v[ds], _NEG_INF)
            s_v[ds] = ns
            take = ns > nbv
            nbv = jnp.where(take, ns, nbv)
            nbj = jnp.where(take, j, nbj)

        return nbv, nbj

    jax.lax.fori_loop(0, _TOP_K, round_body, (bestv0, bestj0))

    plsc.subcore_barrier()
    @pl.when(jnp.logical_and(s_id == 0, c_id == 0))
    def _():
        pltpu.sync_copy(rows_sh, out_hbm)


def _plane(x):
    return jnp.pad(x, (0, _PAD - _N)).reshape(_NS, _W)


@jax.jit
def kernel(loc, conf, iou, priors):
    loc_p = jnp.stack([_plane(loc[:, k]) for k in range(14)])
    conf_p = _plane(conf[:, 1])
    iou_p = _plane(iou[:, 0])
    pri_p = jnp.stack([_plane(priors[:, k]) for k in range(4)])

    planes = pl.pallas_call(
        _decode_kernel,
        out_shape=jax.ShapeDtypeStruct((16, _NS, _W), jnp.float32),
    )(loc_p, conf_p, iou_p, pri_p)

    mesh = plsc.VectorSubcoreMesh(core_axis_name="c", subcore_axis_name="s",
                                  num_cores=2, num_subcores=_NS)
    nms = functools.partial(
        pl.kernel,
        out_type=jax.ShapeDtypeStruct((_TOP_K * 16,), jnp.float32),
        mesh=mesh,
        scratch_types=[
            pltpu.VMEM((_W,), jnp.float32),
            pltpu.VMEM((_W,), jnp.float32),
            pltpu.VMEM((_W,), jnp.float32),
            pltpu.VMEM((_W,), jnp.float32),
            pltpu.VMEM((_W,), jnp.float32),
            pltpu.VMEM((_W,), jnp.float32),
            pltpu.VMEM((_W,), jnp.float32),
            pltpu.VMEM((10 * _W,), jnp.float32),
            pltpu.VMEM((16,), jnp.float32),
            pltpu.VMEM((16 * 16,), jnp.float32),
            pltpu.VMEM((16,), jnp.float32),
            pltpu.VMEM_SHARED((2 * 16 * 16,), jnp.float32),
            pltpu.VMEM_SHARED((_TOP_K * 16,), jnp.float32),
        ],
    )(_nms_sc_kernel)

    out = nms(planes).reshape(_TOP_K, 16)
    return out[:, :15]


# speedup vs baseline: 1.0918x; 1.0879x over previous
import functools

import jax
import jax.numpy as jnp
from jax.experimental import pallas as pl
from jax.experimental.pallas import tpu as pltpu
from jax.experimental.pallas import tpu_sc as plsc

_N = 20000
_TOP_K = 50
_IOU_THR = 0.3
_V0 = 0.1
_V1 = 0.2
_NS = 16
_W = 1280
_PAD = _NS * _W
_NVREG = _W // 16
_NEG_INF = float('-inf')


def _decode_kernel(loc_ref, conf_ref, iou_ref, pri_ref, out_ref):
    pcx = pri_ref[0]
    pcy = pri_ref[1]
    pw = pri_ref[2]
    ph = pri_ref[3]

    rowi = jax.lax.broadcasted_iota(jnp.int32, (_NS, _W), 0)
    coli = jax.lax.broadcasted_iota(jnp.int32, (_NS, _W), 1)
    lin = rowi * _W + coli

    cls = conf_ref[...]
    iouc = jnp.clip(iou_ref[...], 0.0, 1.0)
    scores = jnp.sqrt(cls * iouc)
    out_ref[0] = jnp.where(lin < _N, scores, _NEG_INF)

    cx = pcx + loc_ref[0] * _V0 * pw
    cy = pcy + loc_ref[1] * _V0 * ph
    wx = pw * jnp.exp(loc_ref[2] * _V0) * 0.5
    hy = ph * jnp.exp(loc_ref[3] * _V1) * 0.5
    x1 = cx - wx
    y1 = cy - hy
    x2 = cx + wx
    y2 = cy + hy
    out_ref[1] = x1
    out_ref[2] = y1
    out_ref[3] = x2
    out_ref[4] = y2
    out_ref[5] = (x2 - x1) * (y2 - y1)
    for k in range(5):
        out_ref[6 + 2 * k] = pcx + loc_ref[4 + 2 * k] * _V0 * pw
        out_ref[7 + 2 * k] = pcy + loc_ref[5 + 2 * k] * _V0 * ph


def _nms_sc_kernel(planes_hbm, out_hbm,
                   s_v, s0_v, x1_v, y1_v, x2_v, y2_v, ar_v, lm_v,
                   cm_v, pk_v, cand_v, pub_v, shloc_v, row_v, sh, rows_sh):
    c_id = jax.lax.axis_index("c")
    s_id = jax.lax.axis_index("s")

    pltpu.sync_copy(planes_hbm.at[0, s_id], s_v)
    pltpu.sync_copy(planes_hbm.at[0, s_id], s0_v)
    pltpu.sync_copy(planes_hbm.at[1, s_id], x1_v)
    pltpu.sync_copy(planes_hbm.at[2, s_id], y1_v)
    pltpu.sync_copy(planes_hbm.at[3, s_id], x2_v)
    pltpu.sync_copy(planes_hbm.at[4, s_id], y2_v)
    pltpu.sync_copy(planes_hbm.at[5, s_id], ar_v)
    for k in range(10):
        pltpu.sync_copy(planes_hbm.at[6 + k, s_id],
                        lm_v.at[pl.ds(k * _W, _W)])

    iota16 = jax.lax.iota(jnp.int32, 16)
    zero16 = jnp.zeros((16,), jnp.int32)

    def bfly_max(v):
        for k in (8, 4, 2, 1):
            v = jnp.maximum(v, v[jnp.bitwise_xor(iota16, k)])
        return v

    def bfly_min(v):
        for k in (8, 4, 2, 1):
            v = jnp.minimum(v, v[jnp.bitwise_xor(iota16, k)])
        return v

    def splat_at(ref, base, lane):
        chunk = ref[pl.ds(base, 16)]
        return chunk[lane + zero16]

    for k in range(20):
        pk_v[pl.ds(16 * k, 16)] = jnp.zeros((16,), jnp.float32)

    for r in range(5):
        acc = jnp.full((16,), _NEG_INF, jnp.float32)
        for l in range(16):
            ch = s_v[pl.ds((16 * r + l) * 16, 16)]
            acc = jnp.where(iota16 == l, bfly_max(ch), acc)
        cm_v[pl.ds(16 * r, 16)] = acc

    def mark(lj0):
        mb = jnp.bitwise_and(lj0, -16)
        ml = jnp.bitwise_and(lj0, 15)
        ch = s_v[pl.ds(mb, 16)]
        ch = jnp.where(iota16 == ml, _NEG_INF, ch)
        s_v[pl.ds(mb, 16)] = ch
        jc = jax.lax.shift_right_logical(lj0, 4)
        cb = jnp.bitwise_and(jc, -16)
        cl = jnp.bitwise_and(jc, 15)
        c2 = cm_v[pl.ds(cb, 16)]
        c2 = jnp.where(iota16 == cl, bfly_max(ch), c2)
        cm_v[pl.ds(cb, 16)] = c2

    def rescan():
        bv = cm_v[pl.ds(0, 16)]
        bj = zero16
        for r in range(1, 5):
            v = cm_v[pl.ds(16 * r, 16)]
            take = v > bv
            bv = jnp.where(take, v, bv)
            bj = jnp.where(take, r, bj)
        mvx = bfly_max(bv)
        jcs = bfly_min(jnp.where(bv == mvx, bj * 16 + iota16,
                                 jnp.int32(1 << 30)))
        ch = s_v[pl.ds(jcs[0] * 16, 16)]
        lns = bfly_min(jnp.where(ch == mvx, iota16, jnp.int32(99)))
        clj = jcs * 16 + lns
        base = jnp.bitwise_and(clj[0], -16)
        lane = jnp.bitwise_and(clj[0], 15)
        return (mvx, clj,
                splat_at(x1_v, base, lane), splat_at(y1_v, base, lane),
                splat_at(x2_v, base, lane), splat_at(y2_v, base, lane),
                splat_at(ar_v, base, lane))

    def notkeep(cx1, cy1, cx2, cy2, car, ox1, oy1, ox2, oy2, oar):
        ix1 = jnp.maximum(ox1, cx1)
        iy1 = jnp.maximum(oy1, cy1)
        ix2 = jnp.minimum(ox2, cx2)
        iy2 = jnp.minimum(oy2, cy2)
        inter = (jnp.maximum(ix2 - ix1, 0.0)
                 * jnp.maximum(iy2 - iy1, 0.0))
        union = oar + car - inter
        iouv = inter / jnp.maximum(union, 1e-12)
        return jnp.where(iouv <= _IOU_THR, 0.0, 1.0)

    def check_all(cx1, cy1, cx2, cy2, car):
        acc = jnp.zeros((16,), jnp.float32)
        for k in range(4):
            ds = pl.ds(16 * k, 16)
            acc = jnp.maximum(acc, notkeep(
                cx1, cy1, cx2, cy2, car,
                pk_v[pl.ds(0 + 16 * k, 16)], pk_v[pl.ds(64 + 16 * k, 16)],
                pk_v[pl.ds(128 + 16 * k, 16)], pk_v[pl.ds(192 + 16 * k, 16)],
                pk_v[pl.ds(256 + 16 * k, 16)]))
        return bfly_max(acc)[0] > 0.5

    def write_cand(cs, clj, cx1, cy1, cx2, cy2, car, dirty):
        ncv = jnp.where(iota16 == 0, cs, 0.0)
        ncv = jnp.where(iota16 == 1, clj.astype(jnp.float32), ncv)
        ncv = jnp.where(iota16 == 2, cx1, ncv)
        ncv = jnp.where(iota16 == 3, cy1, ncv)
        ncv = jnp.where(iota16 == 4, cx2, ncv)
        ncv = jnp.where(iota16 == 5, cy2, ncv)
        ncv = jnp.where(iota16 == 6, car, ncv)
        ncv = jnp.where(iota16 == 7, jnp.where(dirty, 1.0, 0.0), ncv)
        cand_v[...] = ncv

    cs0, clj0, a0, b0, c0, d0, e0 = rescan()
    write_cand(cs0, clj0, a0, b0, c0, d0, e0, False)

    def round_body(i, carry):
        cv = cand_v[...]

        gidxf = cv[1] + (s_id * _W).astype(jnp.float32)
        pub = jnp.where(iota16 == 1, gidxf, cv)
        pub = jnp.where(iota16 == 7, s_id.astype(jnp.float32), pub)
        pub_v[...] = pub
        par = jnp.bitwise_and(i, 1) * 256
        pltpu.sync_copy(pub_v, sh.at[pl.ds(par + 16 * s_id, 16)])
        plsc.subcore_barrier()

        pltpu.sync_copy(sh.at[pl.ds(par, 256)], shloc_v)

        def merge(a, b):
            va = a[0]
            vb = b[0]
            cond = jnp.logical_or(
                vb > va, jnp.logical_and(vb == va, b[1] < a[1]))
            return jnp.where(cond, b, a)

        rows = [shloc_v[pl.ds(16 * w, 16)] for w in range(_NS)]
        while len(rows) > 1:
            rows = [merge(rows[2 * k], rows[2 * k + 1])
                    for k in range(len(rows) // 2)]
        best = rows[0]
        bx1 = best[2]
        by1 = best[3]
        bx2 = best[4]
        by2 = best[5]
        barea = best[6]
        w_id = best[7].astype(jnp.int32)

        ib = jnp.bitwise_and(i, -16)
        il = jnp.bitwise_and(i, 15)
        for k, val in enumerate([bx1, by1, bx2, by2, barea]):
            c = pk_v[pl.ds(64 * k + ib, 16)]
            pk_v[pl.ds(64 * k + ib, 16)] = jnp.where(iota16 == il, val, c)

        @pl.when(jnp.logical_and(w_id == s_id, c_id == 0))
        def _():
            ljw = best[1].astype(jnp.int32) - s_id * _W
            wbase = jnp.bitwise_and(ljw, -16)
            wlane = jnp.bitwise_and(ljw, 15)
            row = jnp.where(iota16 == 0, bx1, 0.0)
            row = jnp.where(iota16 == 1, by1, row)
            row = jnp.where(iota16 == 2, bx2, row)
            row = jnp.where(iota16 == 3, by2, row)
            for k in range(10):
                lmv = splat_at(lm_v, k * _W + wbase, wlane)
                row = jnp.where(iota16 == 4 + k, lmv, row)
            row = jnp.where(iota16 == 14, splat_at(s0_v, wbase, wlane), row)
            row_v[...] = row
            pltpu.sync_copy(row_v, rows_sh.at[pl.ds(16 * i, 16)])

        zf = jnp.zeros((16,), jnp.float32)
        nkw = notkeep(cv[2] + zf, cv[3] + zf, cv[4] + zf,
                      cv[5] + zf, cv[6] + zf,
                      bx1, by1, bx2, by2, barea)
        live = cv[0] != _NEG_INF
        dirtyv = jnp.where(live, nkw, zf)
        cand_v[...] = jnp.where(iota16 == 7, dirtyv, cv)

        for _t in range(4):
            cvt = cand_v[...]

            @pl.when(cvt[7] > 0.5)
            def _():
                cvv = cand_v[...]
                mark(cvv[1].astype(jnp.int32))
                cs2, clj2, a1, b1, a2, b2, ar2 = rescan()
                d2 = jnp.logical_and(check_all(a1, b1, a2, b2, ar2),
                                     cs2[0] != _NEG_INF)
                write_cand(cs2, clj2, a1, b1, a2, b2, ar2, d2)

        cvt = cand_v[...]

        @pl.when(cvt[7] > 0.5)
        def _():
            def pk_body(b, carry2):
                pb = jnp.bitwise_and(b, -16)
                pln = jnp.bitwise_and(b, 15)
                ox1 = splat_at(pk_v, pb, pln)
                oy1 = splat_at(pk_v, 64 + pb, pln)
                ox2 = splat_at(pk_v, 128 + pb, pln)
                oy2 = splat_at(pk_v, 192 + pb, pln)
                oar = splat_at(pk_v, 256 + pb, pln)

                def ch_body(j, carry3):
                    dsj = pl.ds(16 * j, 16)
                    nk = notkeep(x1_v[dsj], y1_v[dsj], x2_v[dsj],
                                 y2_v[dsj], ar_v[dsj],
                                 ox1, oy1, ox2, oy2, oar)
                    s_v[dsj] = jnp.where(nk > 0.5, _NEG_INF, s_v[dsj])
                    return carry3

                jax.lax.fori_loop(0, _NVREG, ch_body, 0)
                return carry2

            jax.lax.fori_loop(0, i + 1, pk_body, 0)
            for r in range(5):
                acc = jnp.full((16,), _NEG_INF, jnp.float32)
                for l in range(16):
                    ch = s_v[pl.ds((16 * r + l) * 16, 16)]
                    acc = jnp.where(iota16 == l, bfly_max(ch), acc)
                cm_v[pl.ds(16 * r, 16)] = acc
            cs2, clj2, a1, b1, a2, b2, ar2 = rescan()
            write_cand(cs2, clj2, a1, b1, a2, b2, ar2, False)

        return carry

    jax.lax.fori_loop(0, _TOP_K, round_body, 0)

    plsc.subcore_barrier()
    @pl.when(jnp.logical_and(s_id == 0, c_id == 0))
    def _():
        pltpu.sync_copy(rows_sh, out_hbm.at[...])


def _plane(x):
    return jnp.pad(x, (0, _PAD - _N)).reshape(_NS, _W)


@jax.jit
def kernel(loc, conf, iou, priors):
    loc_p = jnp.stack([_plane(loc[:, k]) for k in range(14)])
    conf_p = _plane(conf[:, 1])
    iou_p = _plane(iou[:, 0])
    pri_p = jnp.stack([_plane(priors[:, k]) for k in range(4)])

    planes = pl.pallas_call(
        _decode_kernel,
        out_shape=jax.ShapeDtypeStruct((16, _NS, _W), jnp.float32),
    )(loc_p, conf_p, iou_p, pri_p)

    mesh = plsc.VectorSubcoreMesh(core_axis_name="c", subcore_axis_name="s",
                                  num_cores=2, num_subcores=_NS)
    nms = functools.partial(
        pl.kernel,
        out_type=jax.ShapeDtypeStruct((_TOP_K * 16,), jnp.float32),
        mesh=mesh,
        scratch_types=[
            pltpu.VMEM((_W,), jnp.float32),
            pltpu.VMEM((_W,), jnp.float32),
            pltpu.VMEM((_W,), jnp.float32),
            pltpu.VMEM((_W,), jnp.float32),
            pltpu.VMEM((_W,), jnp.float32),
            pltpu.VMEM((_W,), jnp.float32),
            pltpu.VMEM((_W,), jnp.float32),
            pltpu.VMEM((10 * _W,), jnp.float32),
            pltpu.VMEM((80,), jnp.float32),
            pltpu.VMEM((5 * 64,), jnp.float32),
            pltpu.VMEM((16,), jnp.float32),
            pltpu.VMEM((16,), jnp.float32),
            pltpu.VMEM((16 * 16,), jnp.float32),
            pltpu.VMEM((16,), jnp.float32),
            pltpu.VMEM_SHARED((2 * 16 * 16,), jnp.float32),
            pltpu.VMEM_SHARED((_TOP_K * 16,), jnp.float32),
        ],
    )(_nms_sc_kernel)

    out = nms(planes).reshape(_TOP_K, 16)
    return out[:, :15]


# speedup vs baseline: 1.2271x; 1.1240x over previous
import functools

import jax
import jax.numpy as jnp
from jax.experimental import pallas as pl
from jax.experimental.pallas import tpu as pltpu
from jax.experimental.pallas import tpu_sc as plsc

_N = 20000
_TOP_K = 50
_IOU_THR = 0.3
_V0 = 0.1
_V1 = 0.2
_NS = 16
_W = 1280
_PAD = _NS * _W
_NVREG = _W // 16
_NEG_INF = float('-inf')


def _decode_kernel(loc_ref, conf_ref, iou_ref, pri_ref, out_ref):
    pcx = pri_ref[0]
    pcy = pri_ref[1]
    pw = pri_ref[2]
    ph = pri_ref[3]

    rowi = jax.lax.broadcasted_iota(jnp.int32, (_NS, _W), 0)
    coli = jax.lax.broadcasted_iota(jnp.int32, (_NS, _W), 1)
    lin = rowi * _W + coli

    cls = conf_ref[...]
    iouc = jnp.clip(iou_ref[...], 0.0, 1.0)
    scores = jnp.sqrt(cls * iouc)
    out_ref[0] = jnp.where(lin < _N, scores, _NEG_INF)

    cx = pcx + loc_ref[0] * _V0 * pw
    cy = pcy + loc_ref[1] * _V0 * ph
    wx = pw * jnp.exp(loc_ref[2] * _V0) * 0.5
    hy = ph * jnp.exp(loc_ref[3] * _V1) * 0.5
    x1 = cx - wx
    y1 = cy - hy
    x2 = cx + wx
    y2 = cy + hy
    out_ref[1] = x1
    out_ref[2] = y1
    out_ref[3] = x2
    out_ref[4] = y2
    out_ref[5] = (x2 - x1) * (y2 - y1)
    for k in range(5):
        out_ref[6 + 2 * k] = pcx + loc_ref[4 + 2 * k] * _V0 * pw
        out_ref[7 + 2 * k] = pcy + loc_ref[5 + 2 * k] * _V0 * ph


def _nms_sc_kernel(planes_hbm, out_hbm,
                   s_v, s0_v, x1_v, y1_v, x2_v, y2_v, ar_v, lm_v,
                   cm_v, pk_v, cand_v, pub_v, shloc_v, row_v, sh, rows_sh):
    c_id = jax.lax.axis_index("c")
    s_id = jax.lax.axis_index("s")

    pltpu.sync_copy(planes_hbm.at[0, s_id], s_v)
    pltpu.sync_copy(planes_hbm.at[0, s_id], s0_v)
    pltpu.sync_copy(planes_hbm.at[1, s_id], x1_v)
    pltpu.sync_copy(planes_hbm.at[2, s_id], y1_v)
    pltpu.sync_copy(planes_hbm.at[3, s_id], x2_v)
    pltpu.sync_copy(planes_hbm.at[4, s_id], y2_v)
    pltpu.sync_copy(planes_hbm.at[5, s_id], ar_v)
    for k in range(10):
        pltpu.sync_copy(planes_hbm.at[6 + k, s_id],
                        lm_v.at[pl.ds(k * _W, _W)])

    iota16 = jax.lax.iota(jnp.int32, 16)
    zero16 = jnp.zeros((16,), jnp.int32)

    def bfly_max(v):
        for k in (8, 4, 2, 1):
            v = jnp.maximum(v, v[jnp.bitwise_xor(iota16, k)])
        return v

    def bfly_min(v):
        for k in (8, 4, 2, 1):
            v = jnp.minimum(v, v[jnp.bitwise_xor(iota16, k)])
        return v

    def splat_at(ref, base, lane):
        chunk = ref[pl.ds(base, 16)]
        return chunk[lane + zero16]

    for k in range(20):
        pk_v[pl.ds(16 * k, 16)] = jnp.zeros((16,), jnp.float32)

    for r in range(5):
        acc = jnp.full((16,), _NEG_INF, jnp.float32)
        for l in range(16):
            ch = s_v[pl.ds((16 * r + l) * 16, 16)]
            acc = jnp.where(iota16 == l, bfly_max(ch), acc)
        cm_v[pl.ds(16 * r, 16)] = acc

    def mark(lj0):
        mb = jnp.bitwise_and(lj0, -16)
        ml = jnp.bitwise_and(lj0, 15)
        ch = s_v[pl.ds(mb, 16)]
        ch = jnp.where(iota16 == ml, _NEG_INF, ch)
        s_v[pl.ds(mb, 16)] = ch
        jc = jax.lax.shift_right_logical(lj0, 4)
        cb = jnp.bitwise_and(jc, -16)
        cl = jnp.bitwise_and(jc, 15)
        c2 = cm_v[pl.ds(cb, 16)]
        c2 = jnp.where(iota16 == cl, bfly_max(ch), c2)
        cm_v[pl.ds(cb, 16)] = c2

    def rescan():
        bv = cm_v[pl.ds(0, 16)]
        bj = zero16
        for r in range(1, 5):
            v = cm_v[pl.ds(16 * r, 16)]
            take = v > bv
            bv = jnp.where(take, v, bv)
            bj = jnp.where(take, r, bj)
        mvx = bfly_max(bv)
        jcs = bfly_min(jnp.where(bv == mvx, bj * 16 + iota16,
                                 jnp.int32(1 << 30)))
        ch = s_v[pl.ds(jcs[0] * 16, 16)]
        lns = bfly_min(jnp.where(ch == mvx, iota16, jnp.int32(99)))
        clj = jcs * 16 + lns
        base = jnp.bitwise_and(clj[0], -16)
        lane = jnp.bitwise_and(clj[0], 15)
        return (mvx, clj,
                splat_at(x1_v, base, lane), splat_at(y1_v, base, lane),
                splat_at(x2_v, base, lane), splat_at(y2_v, base, lane),
                splat_at(ar_v, base, lane))

    def notkeep(cx1, cy1, cx2, cy2, car, ox1, oy1, ox2, oy2, oar):
        ix1 = jnp.maximum(ox1, cx1)
        iy1 = jnp.maximum(oy1, cy1)
        ix2 = jnp.minimum(ox2, cx2)
        iy2 = jnp.minimum(oy2, cy2)
        inter = (jnp.maximum(ix2 - ix1, 0.0)
                 * jnp.maximum(iy2 - iy1, 0.0))
        union = oar + car - inter
        iouv = inter / jnp.maximum(union, 1e-12)
        return jnp.where(iouv <= _IOU_THR, 0.0, 1.0)

    def check_all(cx1, cy1, cx2, cy2, car):
        acc = jnp.zeros((16,), jnp.float32)
        for k in range(4):
            ds = pl.ds(16 * k, 16)
            acc = jnp.maximum(acc, notkeep(
                cx1, cy1, cx2, cy2, car,
                pk_v[pl.ds(0 + 16 * k, 16)], pk_v[pl.ds(64 + 16 * k, 16)],
                pk_v[pl.ds(128 + 16 * k, 16)], pk_v[pl.ds(192 + 16 * k, 16)],
                pk_v[pl.ds(256 + 16 * k, 16)]))
        return bfly_max(acc)[0] > 0.5

    def write_cand(cs, clj, cx1, cy1, cx2, cy2, car, dirty):
        ncv = jnp.where(iota16 == 0, cs, 0.0)
        ncv = jnp.where(iota16 == 1, clj.astype(jnp.float32), ncv)
        ncv = jnp.where(iota16 == 2, cx1, ncv)
        ncv = jnp.where(iota16 == 3, cy1, ncv)
        ncv = jnp.where(iota16 == 4, cx2, ncv)
        ncv = jnp.where(iota16 == 5, cy2, ncv)
        ncv = jnp.where(iota16 == 6, car, ncv)
        ncv = jnp.where(iota16 == 7, jnp.where(dirty, 1.0, 0.0), ncv)
        cand_v[...] = ncv

    cs0, clj0, a0, b0, c0, d0, e0 = rescan()
    write_cand(cs0, clj0, a0, b0, c0, d0, e0, False)

    def round_body(i, carry):
        cv = cand_v[...]

        gidxf = cv[1] + (s_id * _W).astype(jnp.float32)
        pub = jnp.where(iota16 == 1, gidxf, cv)
        pub = jnp.where(iota16 == 7, s_id.astype(jnp.float32), pub)
        pub_v[...] = pub
        par = jnp.bitwise_and(i, 1) * 256
        pltpu.sync_copy(pub_v, sh.at[pl.ds(par + 16 * s_id, 16)])
        plsc.subcore_barrier()

        pltpu.sync_copy(sh.at[pl.ds(par, 256)], shloc_v)

        def merge(a, b):
            va = a[0]
            vb = b[0]
            cond = jnp.logical_or(
                vb > va, jnp.logical_and(vb == va, b[1] < a[1]))
            return jnp.where(cond, b, a)

        rows = [shloc_v[pl.ds(16 * w, 16)] for w in range(_NS)]
        while len(rows) > 1:
            rows = [merge(rows[2 * k], rows[2 * k + 1])
                    for k in range(len(rows) // 2)]
        best = rows[0]
        bx1 = best[2]
        by1 = best[3]
        bx2 = best[4]
        by2 = best[5]
        barea = best[6]
        w_id = best[7].astype(jnp.int32)

        ib = jnp.bitwise_and(i, -16)
        il = jnp.bitwise_and(i, 15)
        for k, val in enumerate([bx1, by1, bx2, by2, barea]):
            c = pk_v[pl.ds(64 * k + ib, 16)]
            pk_v[pl.ds(64 * k + ib, 16)] = jnp.where(iota16 == il, val, c)

        @pl.when(jnp.logical_and(w_id == s_id, c_id == 0))
        def _():
            ljw = best[1].astype(jnp.int32) - s_id * _W
            wbase = jnp.bitwise_and(ljw, -16)
            wlane = jnp.bitwise_and(ljw, 15)
            row = jnp.where(iota16 == 0, bx1, 0.0)
            row = jnp.where(iota16 == 1, by1, row)
            row = jnp.where(iota16 == 2, bx2, row)
            row = jnp.where(iota16 == 3, by2, row)
            for k in range(10):
                lmv = splat_at(lm_v, k * _W + wbase, wlane)
                row = jnp.where(iota16 == 4 + k, lmv, row)
            row = jnp.where(iota16 == 14, splat_at(s0_v, wbase, wlane), row)
            row_v[...] = row
            pltpu.sync_copy(row_v, rows_sh.at[pl.ds(16 * i, 16)])

        zf = jnp.zeros((16,), jnp.float32)
        nkw = notkeep(cv[2] + zf, cv[3] + zf, cv[4] + zf,
                      cv[5] + zf, cv[6] + zf,
                      bx1, by1, bx2, by2, barea)
        live = cv[0] != _NEG_INF
        dirtyv = jnp.where(live, nkw, zf)
        cand_v[...] = jnp.where(iota16 == 7, dirtyv, cv)

        for _t in range(4):
            cvt = cand_v[...]

            @pl.when(cvt[7] > 0.5)
            def _():
                cvv = cand_v[...]
                mark(cvv[1].astype(jnp.int32))
                cs2, clj2, a1, b1, a2, b2, ar2 = rescan()
                d2 = jnp.logical_and(check_all(a1, b1, a2, b2, ar2),
                                     cs2[0] != _NEG_INF)
                write_cand(cs2, clj2, a1, b1, a2, b2, ar2, d2)

        cvt = cand_v[...]

        @pl.when(cvt[7] > 0.5)
        def _():
            def pk_body(b, carry2):
                pb = jnp.bitwise_and(b, -16)
                pln = jnp.bitwise_and(b, 15)
                ox1 = splat_at(pk_v, pb, pln)
                oy1 = splat_at(pk_v, 64 + pb, pln)
                ox2 = splat_at(pk_v, 128 + pb, pln)
                oy2 = splat_at(pk_v, 192 + pb, pln)
                oar = splat_at(pk_v, 256 + pb, pln)

                def ch_body(j, carry3):
                    dsj = pl.ds(16 * j, 16)
                    nk = notkeep(x1_v[dsj], y1_v[dsj], x2_v[dsj],
                                 y2_v[dsj], ar_v[dsj],
                                 ox1, oy1, ox2, oy2, oar)
                    s_v[dsj] = jnp.where(nk > 0.5, _NEG_INF, s_v[dsj])
                    return carry3

                jax.lax.fori_loop(0, _NVREG, ch_body, 0)
                return carry2

            jax.lax.fori_loop(0, i + 1, pk_body, 0)
            for r in range(5):
                acc = jnp.full((16,), _NEG_INF, jnp.float32)
                for l in range(16):
                    ch = s_v[pl.ds((16 * r + l) * 16, 16)]
                    acc = jnp.where(iota16 == l, bfly_max(ch), acc)
                cm_v[pl.ds(16 * r, 16)] = acc
            cs2, clj2, a1, b1, a2, b2, ar2 = rescan()
            write_cand(cs2, clj2, a1, b1, a2, b2, ar2, False)

        return carry

    jax.lax.fori_loop(0, _TOP_K, round_body, 0)

    plsc.subcore_barrier()
    @pl.when(jnp.logical_and(s_id == 0, c_id == 0))
    def _():
        pltpu.sync_copy(rows_sh, out_hbm.at[...])


def _planes(x):
    xt = jnp.pad(x.T, ((0, 0), (0, _PAD - _N)))
    return xt.reshape(x.shape[1], _NS, _W)


@jax.jit
def kernel(loc, conf, iou, priors):
    loc_p = _planes(loc)
    conf_p = _planes(conf)[1]
    iou_p = _planes(iou)[0]
    pri_p = _planes(priors)

    planes = pl.pallas_call(
        _decode_kernel,
        out_shape=jax.ShapeDtypeStruct((16, _NS, _W), jnp.float32),
    )(loc_p, conf_p, iou_p, pri_p)

    mesh = plsc.VectorSubcoreMesh(core_axis_name="c", subcore_axis_name="s",
                                  num_cores=2, num_subcores=_NS)
    nms = functools.partial(
        pl.kernel,
        out_type=jax.ShapeDtypeStruct((_TOP_K * 16,), jnp.float32),
        mesh=mesh,
        scratch_types=[
            pltpu.VMEM((_W,), jnp.float32),
            pltpu.VMEM((_W,), jnp.float32),
            pltpu.VMEM((_W,), jnp.float32),
            pltpu.VMEM((_W,), jnp.float32),
            pltpu.VMEM((_W,), jnp.float32),
            pltpu.VMEM((_W,), jnp.float32),
            pltpu.VMEM((_W,), jnp.float32),
            pltpu.VMEM((10 * _W,), jnp.float32),
            pltpu.VMEM((80,), jnp.float32),
            pltpu.VMEM((5 * 64,), jnp.float32),
            pltpu.VMEM((16,), jnp.float32),
            pltpu.VMEM((16,), jnp.float32),
            pltpu.VMEM((16 * 16,), jnp.float32),
            pltpu.VMEM((16,), jnp.float32),
            pltpu.VMEM_SHARED((2 * 16 * 16,), jnp.float32),
            pltpu.VMEM_SHARED((_TOP_K * 16,), jnp.float32),
        ],
    )(_nms_sc_kernel)

    out = nms(planes).reshape(_TOP_K, 16)
    return out[:, :15]


# speedup vs baseline: 1.3922x; 1.1345x over previous
import functools

import jax
import jax.numpy as jnp
from jax.experimental import pallas as pl
from jax.experimental.pallas import tpu as pltpu
from jax.experimental.pallas import tpu_sc as plsc

_N = 20000
_TOP_K = 50
_IOU_THR = 0.3
_V0 = 0.1
_V1 = 0.2
_NS = 16
_W = 1280
_PAD = _NS * _W
_NVREG = _W // 16
_NEG_INF = float('-inf')


def _decode_kernel(loc_ref, conf_ref, iou_ref, pri_ref, out_ref):
    pcx = pri_ref[0]
    pcy = pri_ref[1]
    pw = pri_ref[2]
    ph = pri_ref[3]

    rowi = jax.lax.broadcasted_iota(jnp.int32, (_NS, _W), 0)
    coli = jax.lax.broadcasted_iota(jnp.int32, (_NS, _W), 1)
    lin = rowi * _W + coli

    cls = conf_ref[...]
    iouc = jnp.clip(iou_ref[...], 0.0, 1.0)
    scores = jnp.sqrt(cls * iouc)
    out_ref[0] = jnp.where(lin < _N, scores, _NEG_INF)

    cx = pcx + loc_ref[0] * _V0 * pw
    cy = pcy + loc_ref[1] * _V0 * ph
    wx = pw * jnp.exp(loc_ref[2] * _V0) * 0.5
    hy = ph * jnp.exp(loc_ref[3] * _V1) * 0.5
    x1 = cx - wx
    y1 = cy - hy
    x2 = cx + wx
    y2 = cy + hy
    out_ref[1] = x1
    out_ref[2] = y1
    out_ref[3] = x2
    out_ref[4] = y2
    out_ref[5] = (x2 - x1) * (y2 - y1)
    for k in range(5):
        out_ref[6 + 2 * k] = pcx + loc_ref[4 + 2 * k] * _V0 * pw
        out_ref[7 + 2 * k] = pcy + loc_ref[5 + 2 * k] * _V0 * ph


def _nms_sc_kernel(planes_hbm, out_hbm,
                   s_v, s0_v, x1_v, y1_v, x2_v, y2_v, ar_v, lm_v,
                   cm_v, pk_v, cand_v, pub_v, shloc_v, row_v, sh, rows_sh,
                   dma_sem):
    c_id = jax.lax.axis_index("c")
    s_id = jax.lax.axis_index("s")

    cps = [
        pltpu.async_copy(planes_hbm.at[0, s_id], s_v, dma_sem),
        pltpu.async_copy(planes_hbm.at[0, s_id], s0_v, dma_sem),
        pltpu.async_copy(planes_hbm.at[1, s_id], x1_v, dma_sem),
        pltpu.async_copy(planes_hbm.at[2, s_id], y1_v, dma_sem),
        pltpu.async_copy(planes_hbm.at[3, s_id], x2_v, dma_sem),
        pltpu.async_copy(planes_hbm.at[4, s_id], y2_v, dma_sem),
        pltpu.async_copy(planes_hbm.at[5, s_id], ar_v, dma_sem),
    ]
    for k in range(10):
        cps.append(pltpu.async_copy(planes_hbm.at[6 + k, s_id],
                                    lm_v.at[pl.ds(k * _W, _W)], dma_sem))
    for cp in cps:
        cp.wait()

    iota16 = jax.lax.iota(jnp.int32, 16)
    zero16 = jnp.zeros((16,), jnp.int32)

    def bfly_max(v):
        for k in (8, 4, 2, 1):
            v = jnp.maximum(v, v[jnp.bitwise_xor(iota16, k)])
        return v

    def bfly_min(v):
        for k in (8, 4, 2, 1):
            v = jnp.minimum(v, v[jnp.bitwise_xor(iota16, k)])
        return v

    def splat_at(ref, base, lane):
        chunk = ref[pl.ds(base, 16)]
        return chunk[lane + zero16]

    for k in range(20):
        pk_v[pl.ds(16 * k, 16)] = jnp.zeros((16,), jnp.float32)

    for r in range(5):
        acc = jnp.full((16,), _NEG_INF, jnp.float32)
        for l in range(16):
            ch = s_v[pl.ds((16 * r + l) * 16, 16)]
            acc = jnp.where(iota16 == l, bfly_max(ch), acc)
        cm_v[pl.ds(16 * r, 16)] = acc

    def mark(lj0):
        mb = jnp.bitwise_and(lj0, -16)
        ml = jnp.bitwise_and(lj0, 15)
        ch = s_v[pl.ds(mb, 16)]
        ch = jnp.where(iota16 == ml, _NEG_INF, ch)
        s_v[pl.ds(mb, 16)] = ch
        jc = jax.lax.shift_right_logical(lj0, 4)
        cb = jnp.bitwise_and(jc, -16)
        cl = jnp.bitwise_and(jc, 15)
        c2 = cm_v[pl.ds(cb, 16)]
        c2 = jnp.where(iota16 == cl, bfly_max(ch), c2)
        cm_v[pl.ds(cb, 16)] = c2

    def rescan():
        bv = cm_v[pl.ds(0, 16)]
        bj = zero16
        for r in range(1, 5):
            v = cm_v[pl.ds(16 * r, 16)]
            take = v > bv
            bv = jnp.where(take, v, bv)
            bj = jnp.where(take, r, bj)
        mvx = bfly_max(bv)
        jcs = bfly_min(jnp.where(bv == mvx, bj * 16 + iota16,
                                 jnp.int32(1 << 30)))
        ch = s_v[pl.ds(jcs[0] * 16, 16)]
        lns = bfly_min(jnp.where(ch == mvx, iota16, jnp.int32(99)))
        clj = jcs * 16 + lns
        base = jnp.bitwise_and(clj[0], -16)
        lane = jnp.bitwise_and(clj[0], 15)
        return (mvx, clj,
                splat_at(x1_v, base, lane), splat_at(y1_v, base, lane),
                splat_at(x2_v, base, lane), splat_at(y2_v, base, lane),
                splat_at(ar_v, base, lane))

    def notkeep(cx1, cy1, cx2, cy2, car, ox1, oy1, ox2, oy2, oar):
        ix1 = jnp.maximum(ox1, cx1)
        iy1 = jnp.maximum(oy1, cy1)
        ix2 = jnp.minimum(ox2, cx2)
        iy2 = jnp.minimum(oy2, cy2)
        inter = (jnp.maximum(ix2 - ix1, 0.0)
                 * jnp.maximum(iy2 - iy1, 0.0))
        union = oar + car - inter
        iouv = inter / jnp.maximum(union, 1e-12)
        return jnp.where(iouv <= _IOU_THR, 0.0, 1.0)

    def check_all(cx1, cy1, cx2, cy2, car):
        acc = jnp.zeros((16,), jnp.float32)
        for k in range(4):
            ds = pl.ds(16 * k, 16)
            acc = jnp.maximum(acc, notkeep(
                cx1, cy1, cx2, cy2, car,
                pk_v[pl.ds(0 + 16 * k, 16)], pk_v[pl.ds(64 + 16 * k, 16)],
                pk_v[pl.ds(128 + 16 * k, 16)], pk_v[pl.ds(192 + 16 * k, 16)],
                pk_v[pl.ds(256 + 16 * k, 16)]))
        return bfly_max(acc)[0] > 0.5

    def write_cand(cs, clj, cx1, cy1, cx2, cy2, car, dirty):
        ncv = jnp.where(iota16 == 0, cs, 0.0)
        ncv = jnp.where(iota16 == 1, clj.astype(jnp.float32), ncv)
        ncv = jnp.where(iota16 == 2, cx1, ncv)
        ncv = jnp.where(iota16 == 3, cy1, ncv)
        ncv = jnp.where(iota16 == 4, cx2, ncv)
        ncv = jnp.where(iota16 == 5, cy2, ncv)
        ncv = jnp.where(iota16 == 6, car, ncv)
        ncv = jnp.where(iota16 == 7, jnp.where(dirty, 1.0, 0.0), ncv)
        cand_v[...] = ncv

    cs0, clj0, a0, b0, c0, d0, e0 = rescan()
    write_cand(cs0, clj0, a0, b0, c0, d0, e0, False)

    def round_body(i, carry):
        cv = cand_v[...]

        gidxf = cv[1] + (s_id * _W).astype(jnp.float32)
        pub = jnp.where(iota16 == 1, gidxf, cv)
        pub = jnp.where(iota16 == 7, s_id.astype(jnp.float32), pub)
        pub_v[...] = pub
        par = jnp.bitwise_and(i, 1) * 256
        pltpu.sync_copy(pub_v, sh.at[pl.ds(par + 16 * s_id, 16)])
        plsc.subcore_barrier()

        pltpu.sync_copy(sh.at[pl.ds(par, 256)], shloc_v)

        def merge(a, b):
            va = a[0]
            vb = b[0]
            cond = jnp.logical_or(
                vb > va, jnp.logical_and(vb == va, b[1] < a[1]))
            return jnp.where(cond, b, a)

        rows = [shloc_v[pl.ds(16 * w, 16)] for w in range(_NS)]
        while len(rows) > 1:
            rows = [merge(rows[2 * k], rows[2 * k + 1])
                    for k in range(len(rows) // 2)]
        best = rows[0]
        bx1 = best[2]
        by1 = best[3]
        bx2 = best[4]
        by2 = best[5]
        barea = best[6]
        w_id = best[7].astype(jnp.int32)

        ib = jnp.bitwise_and(i, -16)
        il = jnp.bitwise_and(i, 15)
        for k, val in enumerate([bx1, by1, bx2, by2, barea]):
            c = pk_v[pl.ds(64 * k + ib, 16)]
            pk_v[pl.ds(64 * k + ib, 16)] = jnp.where(iota16 == il, val, c)

        @pl.when(jnp.logical_and(w_id == s_id, c_id == 0))
        def _():
            ljw = best[1].astype(jnp.int32) - s_id * _W
            wbase = jnp.bitwise_and(ljw, -16)
            wlane = jnp.bitwise_and(ljw, 15)
            row = jnp.where(iota16 == 0, bx1, 0.0)
            row = jnp.where(iota16 == 1, by1, row)
            row = jnp.where(iota16 == 2, bx2, row)
            row = jnp.where(iota16 == 3, by2, row)
            for k in range(10):
                lmv = splat_at(lm_v, k * _W + wbase, wlane)
                row = jnp.where(iota16 == 4 + k, lmv, row)
            row = jnp.where(iota16 == 14, splat_at(s0_v, wbase, wlane), row)
            row_v[...] = row
            pltpu.sync_copy(row_v, rows_sh.at[pl.ds(16 * i, 16)])

        zf = jnp.zeros((16,), jnp.float32)
        nkw = notkeep(cv[2] + zf, cv[3] + zf, cv[4] + zf,
                      cv[5] + zf, cv[6] + zf,
                      bx1, by1, bx2, by2, barea)
        live = cv[0] != _NEG_INF
        dirtyv = jnp.where(live, nkw, zf)
        cand_v[...] = jnp.where(iota16 == 7, dirtyv, cv)

        for _t in range(4):
            cvt = cand_v[...]

            @pl.when(cvt[7] > 0.5)
            def _():
                cvv = cand_v[...]
                mark(cvv[1].astype(jnp.int32))
                cs2, clj2, a1, b1, a2, b2, ar2 = rescan()
                d2 = jnp.logical_and(check_all(a1, b1, a2, b2, ar2),
                                     cs2[0] != _NEG_INF)
                write_cand(cs2, clj2, a1, b1, a2, b2, ar2, d2)

        cvt = cand_v[...]

        @pl.when(cvt[7] > 0.5)
        def _():
            def pk_body(b, carry2):
                pb = jnp.bitwise_and(b, -16)
                pln = jnp.bitwise_and(b, 15)
                ox1 = splat_at(pk_v, pb, pln)
                oy1 = splat_at(pk_v, 64 + pb, pln)
                ox2 = splat_at(pk_v, 128 + pb, pln)
                oy2 = splat_at(pk_v, 192 + pb, pln)
                oar = splat_at(pk_v, 256 + pb, pln)

                def ch_body(j, carry3):
                    dsj = pl.ds(16 * j, 16)
                    nk = notkeep(x1_v[dsj], y1_v[dsj], x2_v[dsj],
                                 y2_v[dsj], ar_v[dsj],
                                 ox1, oy1, ox2, oy2, oar)
                    s_v[dsj] = jnp.where(nk > 0.5, _NEG_INF, s_v[dsj])
                    return carry3

                jax.lax.fori_loop(0, _NVREG, ch_body, 0)
                return carry2

            jax.lax.fori_loop(0, i + 1, pk_body, 0)
            for r in range(5):
                acc = jnp.full((16,), _NEG_INF, jnp.float32)
                for l in range(16):
                    ch = s_v[pl.ds((16 * r + l) * 16, 16)]
                    acc = jnp.where(iota16 == l, bfly_max(ch), acc)
                cm_v[pl.ds(16 * r, 16)] = acc
            cs2, clj2, a1, b1, a2, b2, ar2 = rescan()
            write_cand(cs2, clj2, a1, b1, a2, b2, ar2, False)

        return carry

    jax.lax.fori_loop(0, _TOP_K, round_body, 0)

    plsc.subcore_barrier()
    @pl.when(jnp.logical_and(s_id == 0, c_id == 0))
    def _():
        pltpu.sync_copy(rows_sh, out_hbm.at[...])


def _planes(x):
    xt = jnp.pad(x.T, ((0, 0), (0, _PAD - _N)))
    return xt.reshape(x.shape[1], _NS, _W)


@jax.jit
def kernel(loc, conf, iou, priors):
    loc_p = _planes(loc)
    conf_p = _planes(conf)[1]
    iou_p = _planes(iou)[0]
    pri_p = _planes(priors)

    planes = pl.pallas_call(
        _decode_kernel,
        out_shape=jax.ShapeDtypeStruct((16, _NS, _W), jnp.float32),
    )(loc_p, conf_p, iou_p, pri_p)

    mesh = plsc.VectorSubcoreMesh(core_axis_name="c", subcore_axis_name="s",
                                  num_cores=2, num_subcores=_NS)
    nms = functools.partial(
        pl.kernel,
        out_type=jax.ShapeDtypeStruct((_TOP_K * 16,), jnp.float32),
        mesh=mesh,
        scratch_types=[
            pltpu.VMEM((_W,), jnp.float32),
            pltpu.VMEM((_W,), jnp.float32),
            pltpu.VMEM((_W,), jnp.float32),
            pltpu.VMEM((_W,), jnp.float32),
            pltpu.VMEM((_W,), jnp.float32),
            pltpu.VMEM((_W,), jnp.float32),
            pltpu.VMEM((_W,), jnp.float32),
            pltpu.VMEM((10 * _W,), jnp.float32),
            pltpu.VMEM((80,), jnp.float32),
            pltpu.VMEM((5 * 64,), jnp.float32),
            pltpu.VMEM((16,), jnp.float32),
            pltpu.VMEM((16,), jnp.float32),
            pltpu.VMEM((16 * 16,), jnp.float32),
            pltpu.VMEM((16,), jnp.float32),
            pltpu.VMEM_SHARED((2 * 16 * 16,), jnp.float32),
            pltpu.VMEM_SHARED((_TOP_K * 16,), jnp.float32),
            pltpu.SemaphoreType.DMA,
        ],
    )(_nms_sc_kernel)

    out = nms(planes).reshape(_TOP_K, 16)
    return out[:, :15]
